# baseline (speedup 1.0000x reference)
import jax
import jax.numpy as jnp
from jax.experimental import pallas as pl
from jax.experimental.pallas import tpu as pltpu

B, C, L, D = 16384, 26, 128, 64
TAU = 0.07
MARGIN = 0.3

BB = 512
NB = B // BB


def _body(logc_ref, labc_ref, tlt_ref, tft_ref, pro_ref, out_ref):
    c = pl.program_id(1)

    x = logc_ref[0, 0]
    y = labc_ref[0, 0]

    bce = jnp.maximum(x, 0.0) - x * y + jnp.log1p(jnp.exp(-jnp.abs(x)))

    t = tlt_ref[0]
    mxr = jnp.max(t, axis=1)
    rej = jnp.maximum(jax.nn.sigmoid(mxr) - MARGIN, 0.0)[None, :]

    p = pro_ref[...]
    pinv = 1.0 / jnp.maximum(
        jnp.sqrt(jnp.sum(p * p, axis=1, keepdims=True)), 1e-12)
    pn = p * pinv
    f = tft_ref[0]
    sqv = jax.lax.dot_general(jnp.ones((1, D), jnp.float32), f * f,
                              (((1,), (1,)), ((), ())),
                              preferred_element_type=jnp.float32)
    finv = 1.0 / jnp.maximum(jnp.sqrt(sqv), 1e-12)
    St = jax.lax.dot_general(pn, f, (((1,), (1,)), ((), ())),
                             preferred_element_type=jnp.float32)
    St = St * (finv * (1.0 / TAU))
    m = jnp.max(St, axis=0, keepdims=True)
    lse = m + jnp.log(jnp.sum(jnp.exp(St - m), axis=0, keepdims=True))
    row = jax.lax.broadcasted_iota(jnp.int32, (C, BB), 0)
    diag = jnp.sum(jnp.where(row == c, St, 0.0), axis=0, keepdims=True)
    psc = lse - diag

    contrib = (bce + jnp.where(y > 0.0, psc, rej))[0]

    @pl.when(c == 0)
    def _init():
        out_ref[...] = contrib

    @pl.when(c != 0)
    def _acc():
        out_ref[...] = out_ref[...] + contrib


def kernel(logits, total_cls_logits, total_cls_feature, labels, prototypes):
    logc = logits.T.reshape(C, NB, 1, BB)
    labc = labels.T.reshape(C, NB, 1, BB)
    grid = (NB, C)
    out = pl.pallas_call(
        _body,
        grid=grid,
        in_specs=[
            pl.BlockSpec((1, 1, 1, BB), lambda i, c: (c, i, 0, 0)),
            pl.BlockSpec((1, 1, 1, BB), lambda i, c: (c, i, 0, 0)),
            pl.BlockSpec((1, BB, L), lambda i, c: (c, i, 0)),
            pl.BlockSpec((1, BB, D), lambda i, c: (c, i, 0)),
            pl.BlockSpec((C, D), lambda i, c: (0, 0)),
        ],
        out_specs=pl.BlockSpec((BB,), lambda i, c: (i,)),
        out_shape=jax.ShapeDtypeStruct((B,), jnp.float32),
    )(logc, labc, total_cls_logits, total_cls_feature, prototypes)
    return out
